# Initial kernel scaffold; baseline (speedup 1.0000x reference)
import jax
import jax.numpy as jnp
from jax.experimental import pallas as pl


def kernel(x, Wqkv, bqkv, Wout, bout):
    raise NotImplementedError("write your pallas kernel here")


# speedup vs baseline: 60.4667x; 60.4667x over previous
import jax
import jax.numpy as jnp
from jax.experimental import pallas as pl

DIM = 1024
HEADS = 16
HD = 64
SEQ = 2048
KTOP = 64
QB = 256
SCALE = 0.125


def _qkv_kernel(x_ref, wq_ref, wk_ref, wv_ref, bq_ref, bk_ref, bv_ref,
                q_ref, k_ref, v_ref):
    x = x_ref[...]
    dims = (((1,), (1,)), ((), ()))
    q = jax.lax.dot_general(x, wq_ref[...], dims,
                            preferred_element_type=jnp.float32) + bq_ref[0]
    q_ref[0] = q * SCALE
    k_ref[0] = jax.lax.dot_general(x, wk_ref[...], dims,
                                   preferred_element_type=jnp.float32) + bk_ref[0]
    v_ref[0] = jax.lax.dot_general(x, wv_ref[...], dims,
                                   preferred_element_type=jnp.float32) + bv_ref[0]


def _attn_kernel(q_ref, k_ref, v_ref, woT_ref, bo_ref, o_ref):
    h = pl.program_id(1)
    s = jax.lax.dot_general(
        q_ref[0], k_ref[0], (((1,), (1,)), ((), ())),
        preferred_element_type=jnp.float32,
    )

    key = jax.lax.bitcast_convert_type(s, jnp.int32)
    key = jnp.where(key >= 0, key, key ^ jnp.int32(0x7FFFFFFF))

    cnt = jnp.sum((key >= 0).astype(jnp.float32), axis=1, keepdims=True)
    prefix = jnp.where(cnt >= KTOP, jnp.int32(0), jnp.int32(-2147483648))
    for bit in range(30, -1, -1):
        trial = prefix | jnp.int32(1 << bit)
        cnt = jnp.sum((key >= trial).astype(jnp.float32), axis=1, keepdims=True)
        prefix = jnp.where(cnt >= KTOP, trial, prefix)

    mask = key >= prefix
    m = jnp.max(s, axis=1, keepdims=True)
    p = jnp.where(mask, jnp.exp(s - m), 0.0)
    z = jnp.sum(p, axis=1, keepdims=True)
    av = jax.lax.dot_general(
        p, v_ref[0], (((1,), (0,)), ((), ())),
        preferred_element_type=jnp.float32,
    )
    av = av / z
    contrib = jax.lax.dot_general(
        av, woT_ref[...], (((1,), (0,)), ((), ())),
        preferred_element_type=jnp.float32,
    )

    @pl.when(h == 0)
    def _init():
        o_ref[...] = bo_ref[...] + contrib

    @pl.when(h != 0)
    def _acc():
        o_ref[...] += contrib


def kernel(x, Wqkv, bqkv, Wout, bout):
    x2 = x[0]
    b3 = bqkv.reshape(3 * HEADS, 1, HD)
    bo2 = bout.reshape(1, DIM)
    woT = Wout.T

    hdstruct = jax.ShapeDtypeStruct((HEADS, SEQ, HD), jnp.float32)
    q, k, v = pl.pallas_call(
        _qkv_kernel,
        grid=(SEQ // QB, HEADS),
        in_specs=[
            pl.BlockSpec((QB, DIM), lambda i, h: (i, 0)),
            pl.BlockSpec((HD, DIM), lambda i, h: (h, 0)),
            pl.BlockSpec((HD, DIM), lambda i, h: (HEADS + h, 0)),
            pl.BlockSpec((HD, DIM), lambda i, h: (2 * HEADS + h, 0)),
            pl.BlockSpec((1, 1, HD), lambda i, h: (h, 0, 0)),
            pl.BlockSpec((1, 1, HD), lambda i, h: (HEADS + h, 0, 0)),
            pl.BlockSpec((1, 1, HD), lambda i, h: (2 * HEADS + h, 0, 0)),
        ],
        out_specs=[
            pl.BlockSpec((1, QB, HD), lambda i, h: (h, i, 0)),
            pl.BlockSpec((1, QB, HD), lambda i, h: (h, i, 0)),
            pl.BlockSpec((1, QB, HD), lambda i, h: (h, i, 0)),
        ],
        out_shape=[hdstruct, hdstruct, hdstruct],
    )(x2, Wqkv, Wqkv, Wqkv, b3, b3, b3)

    out = pl.pallas_call(
        _attn_kernel,
        grid=(SEQ // QB, HEADS),
        in_specs=[
            pl.BlockSpec((1, QB, HD), lambda i, h: (h, i, 0)),
            pl.BlockSpec((1, SEQ, HD), lambda i, h: (h, 0, 0)),
            pl.BlockSpec((1, SEQ, HD), lambda i, h: (h, 0, 0)),
            pl.BlockSpec((HD, DIM), lambda i, h: (h, 0)),
            pl.BlockSpec((1, DIM), lambda i, h: (0, 0)),
        ],
        out_specs=pl.BlockSpec((QB, DIM), lambda i, h: (i, 0)),
        out_shape=jax.ShapeDtypeStruct((SEQ, DIM), jnp.float32),
    )(q, k, v, woT, bo2)

    return out[None]
